# Initial kernel scaffold; baseline (speedup 1.0000x reference)
import jax
import jax.numpy as jnp
from jax.experimental import pallas as pl


def kernel(idxes, table):
    raise NotImplementedError("write your pallas kernel here")


# speedup vs baseline: 1.0929x; 1.0929x over previous
import functools

import jax
import jax.numpy as jnp
from jax import lax
from jax.experimental import pallas as pl
from jax.experimental.pallas import tpu as pltpu
from jax.experimental.pallas import tpu_sc as plsc

_NC = 2
_NS = 16
_NW = _NC * _NS

_GRP = 128
_CHUNK = 1024


def kernel(idxes, table):
    B, H = idxes.shape
    V, D = table.shape
    N = B * H

    idx2d = idxes.reshape(N // _GRP, _GRP).astype(jnp.int32)

    b_per_w = N // _NW
    n_chunks = b_per_w // _CHUNK
    grp_per_chunk = _CHUNK // _GRP
    grp_rows_per_w = b_per_w // _GRP

    mesh = plsc.VectorSubcoreMesh(core_axis_name="c", subcore_axis_name="s")

    @functools.partial(
        pl.kernel,
        mesh=mesh,
        out_type=jax.ShapeDtypeStruct((N, D), jnp.float32),
        compiler_params=pltpu.CompilerParams(use_tc_tiling_on_sc=False),
        scratch_types=[
            pltpu.VMEM((grp_per_chunk, _GRP), jnp.int32),
            pltpu.VMEM((_CHUNK, D), jnp.float32),
            pltpu.SemaphoreType.DMA,
        ],
    )
    def run(idx_hbm, table_hbm, out_hbm, idx_v, rows_v, sem):
        wid = lax.axis_index("s") * _NC + lax.axis_index("c")
        base = wid * b_per_w
        grp_base = wid * grp_rows_per_w

        def body(i, carry):
            pltpu.sync_copy(
                idx_hbm.at[pl.ds(grp_base + i * grp_per_chunk, grp_per_chunk)],
                idx_v,
            )
            copies = [
                pltpu.async_copy(
                    table_hbm.at[idx_v.at[g]],
                    rows_v.at[pl.ds(g * _GRP, _GRP)],
                    sem,
                )
                for g in range(grp_per_chunk)
            ]
            for c in copies:
                c.wait()
            pltpu.sync_copy(rows_v, out_hbm.at[pl.ds(base + i * _CHUNK, _CHUNK)])
            return carry

        lax.fori_loop(0, n_chunks, body, 0)

    out = run(idx2d, table)
    return out.reshape(B, H, D)


# speedup vs baseline: 1.1112x; 1.0167x over previous
import functools

import jax
import jax.numpy as jnp
from jax import lax
from jax.experimental import pallas as pl
from jax.experimental.pallas import tpu as pltpu
from jax.experimental.pallas import tpu_sc as plsc

_NC = 2
_NS = 16
_NW = _NC * _NS

_GRP = 128
_CHUNK = 1280
_NBUF = 2


def kernel(idxes, table):
    B, H = idxes.shape
    V, D = table.shape
    N = B * H

    idx2d = idxes.reshape(N // _GRP, _GRP).astype(jnp.int32)

    b_per_w = N // _NW
    T = b_per_w // _CHUNK
    G = _CHUNK // _GRP
    grp_rows_per_w = b_per_w // _GRP
    assert b_per_w * _NW == N and T * _CHUNK == b_per_w and T % 2 == 0 and T >= 6

    mesh = plsc.VectorSubcoreMesh(core_axis_name="c", subcore_axis_name="s")

    @functools.partial(
        pl.kernel,
        mesh=mesh,
        out_type=jax.ShapeDtypeStruct((N, D), jnp.float32),
        compiler_params=pltpu.CompilerParams(use_tc_tiling_on_sc=False),
        scratch_types=[
            pltpu.VMEM((_NBUF, G, _GRP), jnp.int32),
            pltpu.VMEM((_NBUF, _CHUNK, D), jnp.float32),
            pltpu.SemaphoreType.DMA,
            pltpu.SemaphoreType.DMA,
            pltpu.SemaphoreType.DMA,
            pltpu.SemaphoreType.DMA,
            pltpu.SemaphoreType.DMA,
            pltpu.SemaphoreType.DMA,
        ],
    )
    def run(idx_hbm, table_hbm, out_hbm, idx_v, rows_v,
            si0, si1, sg0, sg1, so0, so1):
        semi = (si0, si1)
        semg = (sg0, sg1)
        semo = (so0, so1)
        wid = lax.axis_index("s") * _NC + lax.axis_index("c")
        base = wid * b_per_w
        grp_base = wid * grp_rows_per_w

        def fire_idx(c, b):
            pltpu.async_copy(
                idx_hbm.at[pl.ds(grp_base + c * G, G)], idx_v.at[b], semi[b])

        def wait_idx(b):
            pltpu.make_async_copy(
                idx_hbm.at[pl.ds(0, G)], idx_v.at[b], semi[b]).wait()

        def fire_gathers(b):
            for g in range(G):
                pltpu.async_copy(
                    table_hbm.at[idx_v.at[b, g]],
                    rows_v.at[b].at[pl.ds(g * _GRP, _GRP)],
                    semg[b])

        def wait_gathers(b):
            pltpu.make_async_copy(
                table_hbm.at[pl.ds(0, _CHUNK)], rows_v.at[b], semg[b]).wait()

        def fire_out(c, b):
            pltpu.async_copy(
                rows_v.at[b], out_hbm.at[pl.ds(base + c * _CHUNK, _CHUNK)],
                semo[b])

        def wait_out(b):
            pltpu.make_async_copy(
                rows_v.at[b], out_hbm.at[pl.ds(0, _CHUNK)], semo[b]).wait()

        fire_idx(0, 0)
        fire_idx(1, 1)
        wait_idx(0)
        fire_gathers(0)
        wait_idx(1)
        fire_gathers(1)
        wait_gathers(0)
        fire_out(0, 0)
        fire_idx(2, 0)

        def step(c, b, pb):
            wait_idx(b)
            wait_out(b)
            fire_gathers(b)
            wait_gathers(pb)
            fire_out(c - 1, pb)
            fire_idx(c + 1, pb)

        def body(g, carry):
            step(2 * g, 0, 1)
            step(2 * g + 1, 1, 0)
            return carry

        lax.fori_loop(1, T // 2 - 1, body, 0)

        step(T - 2, 0, 1)
        wait_idx(1)
        wait_out(1)
        fire_gathers(1)
        wait_gathers(0)
        fire_out(T - 2, 0)
        wait_gathers(1)
        fire_out(T - 1, 1)
        wait_out(0)
        wait_out(1)

    out = run(idx2d, table)
    return out.reshape(B, H, D)


# speedup vs baseline: 1.4601x; 1.3140x over previous
import functools

import jax
import jax.numpy as jnp
from jax import lax
from jax.experimental import pallas as pl
from jax.experimental.pallas import tpu as pltpu
from jax.experimental.pallas import tpu_sc as plsc

_NC = 2
_NS = 16
_NW = _NC * _NS

_GRP = 128


def kernel(idxes, table):
    B, H = idxes.shape
    V, D = table.shape

    idx_t = idxes.T

    BW = B // _NW
    G = BW // _GRP
    LANES = 16
    assert BW * _NW == B and G * _GRP == BW and H % 2 == 0 and H >= 6

    mesh = plsc.VectorSubcoreMesh(core_axis_name="c", subcore_axis_name="s")

    @functools.partial(
        pl.kernel,
        mesh=mesh,
        out_type=jax.ShapeDtypeStruct((H, D, B), jnp.float32),
        compiler_params=pltpu.CompilerParams(
            use_tc_tiling_on_sc=False, needs_layout_passes=False),
        scratch_types=[
            pltpu.VMEM((G, _GRP), jnp.int32),
            pltpu.VMEM((G, _GRP), jnp.int32),
            pltpu.VMEM((BW, D), jnp.float32),
            pltpu.VMEM((BW, D), jnp.float32),
            pltpu.VMEM((1, D, BW), jnp.float32),
            pltpu.VMEM((1, D, BW), jnp.float32),
            pltpu.SemaphoreType.DMA,
            pltpu.SemaphoreType.DMA,
            pltpu.SemaphoreType.DMA,
            pltpu.SemaphoreType.DMA,
            pltpu.SemaphoreType.DMA,
            pltpu.SemaphoreType.DMA,
        ],
    )
    def run(idx_hbm, table_hbm, out_hbm,
            idx0, idx1, rows0, rows1, rt0, rt1,
            si0, si1, sg0, sg1, so0, so1):
        idx_v = (idx0, idx1)
        rows_v = (rows0, rows1)
        rt_v = (rt0, rt1)
        semi = (si0, si1)
        semg = (sg0, sg1)
        semo = (so0, so1)

        wid = lax.axis_index("s") * _NC + lax.axis_index("c")
        b0 = wid * BW

        def fire_idx(h, nb):
            for g in range(G):
                pltpu.async_copy(
                    idx_hbm.at[pl.ds(h, 1), pl.ds(b0 + g * _GRP, _GRP)],
                    idx_v[nb].at[pl.ds(g, 1)],
                    semi[nb])

        def wait_idx(nb):
            pltpu.make_async_copy(
                idx_hbm.at[pl.ds(0, G), pl.ds(0, _GRP)], idx_v[nb],
                semi[nb]).wait()

        def fire_gathers(nb):
            for g in range(G):
                pltpu.async_copy(
                    table_hbm.at[idx_v[nb].at[g]],
                    rows_v[nb].at[pl.ds(g * _GRP, _GRP)],
                    semg[nb])

        def wait_gathers(nb):
            pltpu.make_async_copy(
                table_hbm.at[pl.ds(0, BW)], rows_v[nb], semg[nb]).wait()

        def transpose(nb):
            rows = rows_v[nb]
            rt = rt_v[nb]

            def tbody(lb, carry):
                row_ids = lb * LANES + jnp.arange(LANES, dtype=jnp.int32)
                for c in range(D):
                    col_ids = jnp.full((LANES,), c, dtype=jnp.int32)
                    v = plsc.load_gather(rows, [row_ids, col_ids])
                    rt[0, c, pl.ds(lb * LANES, LANES)] = v
                return carry

            lax.fori_loop(0, BW // LANES, tbody, 0)

        def fire_out(h, nb):
            pltpu.async_copy(
                rt_v[nb], out_hbm.at[pl.ds(h, 1), :, pl.ds(b0, BW)],
                semo[nb])

        def wait_out(nb):
            pltpu.make_async_copy(
                rt_v[nb], out_hbm.at[pl.ds(0, 1), :, pl.ds(0, BW)],
                semo[nb]).wait()

        def step(h, b, pb, first_use):
            wait_idx(b)
            fire_gathers(b)
            wait_gathers(pb)
            if not first_use:
                wait_out(pb)
            transpose(pb)
            fire_out(h - 1, pb)
            fire_idx(h + 1, pb)

        fire_idx(0, 0)
        fire_idx(1, 1)
        wait_idx(0)
        fire_gathers(0)
        step(1, 1, 0, first_use=True)
        step(2, 0, 1, first_use=True)
        step(3, 1, 0, first_use=False)

        def body(g, carry):
            h = 2 * g
            step(h, 0, 1, first_use=False)
            step(h + 1, 1, 0, first_use=False)
            return carry

        lax.fori_loop(2, H // 2 - 1, body, 0)

        step(H - 2, 0, 1, first_use=False)
        wait_idx(1)
        fire_gathers(1)
        wait_gathers(0)
        wait_out(0)
        transpose(0)
        fire_out(H - 2, 0)
        wait_gathers(1)
        wait_out(1)
        transpose(1)
        fire_out(H - 1, 1)
        wait_out(0)
        wait_out(1)

    out = run(idx_t, table)
    return jnp.transpose(out, (2, 0, 1))


# speedup vs baseline: 1.8667x; 1.2785x over previous
import functools

import jax
import jax.numpy as jnp
from jax import lax
from jax.experimental import pallas as pl
from jax.experimental.pallas import tpu as pltpu
from jax.experimental.pallas import tpu_sc as plsc

_NC = 2
_NS = 16
_NW = _NC * _NS

_GRP = 128


def kernel(idxes, table):
    B, H = idxes.shape
    V, D = table.shape

    idx_t = idxes.T

    BW = B // _NW
    G = BW // _GRP
    LANES = 16
    assert BW * _NW == B and G * _GRP == BW and H % 2 == 0 and H >= 6

    mesh = plsc.VectorSubcoreMesh(core_axis_name="c", subcore_axis_name="s")

    @functools.partial(
        pl.kernel,
        mesh=mesh,
        out_type=jax.ShapeDtypeStruct((H, D, B), jnp.float32),
        compiler_params=pltpu.CompilerParams(
            use_tc_tiling_on_sc=False, needs_layout_passes=False),
        scratch_types=[
            pltpu.VMEM((G, _GRP), jnp.int32),
            pltpu.VMEM((G, _GRP), jnp.int32),
            pltpu.VMEM((BW, D), jnp.float32),
            pltpu.VMEM((BW, D), jnp.float32),
            pltpu.VMEM((1, D, BW), jnp.float32),
            pltpu.VMEM((1, D, BW), jnp.float32),
            pltpu.SemaphoreType.DMA,
            pltpu.SemaphoreType.DMA,
            pltpu.SemaphoreType.DMA,
            pltpu.SemaphoreType.DMA,
            pltpu.SemaphoreType.DMA,
            pltpu.SemaphoreType.DMA,
        ],
    )
    def run(idx_hbm, table_hbm, out_hbm,
            idx0, idx1, rows0, rows1, rt0, rt1,
            si0, si1, sg0, sg1, so0, so1):
        idx_v = (idx0, idx1)
        rows_v = (rows0, rows1)
        rt_v = (rt0, rt1)
        semi = (si0, si1)
        semg = (sg0, sg1)
        semo = (so0, so1)

        wid = lax.axis_index("s") * _NC + lax.axis_index("c")
        b0 = wid * BW

        def fire_idx(h, nb):
            for g in range(G):
                pltpu.async_copy(
                    idx_hbm.at[pl.ds(h, 1), pl.ds(b0 + g * _GRP, _GRP)],
                    idx_v[nb].at[pl.ds(g, 1)],
                    semi[nb])

        def wait_idx(nb):
            pltpu.make_async_copy(
                idx_hbm.at[pl.ds(0, G), pl.ds(0, _GRP)], idx_v[nb],
                semi[nb]).wait()

        def fire_gathers(nb):
            for g in range(G):
                pltpu.async_copy(
                    table_hbm.at[idx_v[nb].at[g]],
                    rows_v[nb].at[pl.ds(g * _GRP, _GRP)],
                    semg[nb])

        def wait_gathers(nb):
            pltpu.make_async_copy(
                table_hbm.at[pl.ds(0, BW)], rows_v[nb], semg[nb]).wait()

        lane_iota = jnp.arange(LANES, dtype=jnp.int32)
        col_ids = [jnp.full((LANES,), c, dtype=jnp.int32) for c in range(D)]

        def transpose(nb):
            rows = rows_v[nb]
            rt = rt_v[nb]

            @plsc.parallel_loop(0, BW // LANES)
            def tbody(lb):
                row_ids = lb * LANES + lane_iota
                for c in range(D):
                    v = plsc.load_gather(rows, [row_ids, col_ids[c]])
                    rt[0, c, pl.ds(lb * LANES, LANES)] = v

        def fire_out(h, nb):
            pltpu.async_copy(
                rt_v[nb], out_hbm.at[pl.ds(h, 1), :, pl.ds(b0, BW)],
                semo[nb])

        def wait_out(nb):
            pltpu.make_async_copy(
                rt_v[nb], out_hbm.at[pl.ds(0, 1), :, pl.ds(0, BW)],
                semo[nb]).wait()

        def step(h, b, pb, first_use):
            wait_idx(b)
            fire_gathers(b)
            wait_gathers(pb)
            if not first_use:
                wait_out(pb)
            transpose(pb)
            fire_out(h - 1, pb)
            fire_idx(h + 1, pb)

        fire_idx(0, 0)
        fire_idx(1, 1)
        wait_idx(0)
        fire_gathers(0)
        step(1, 1, 0, first_use=True)
        step(2, 0, 1, first_use=True)
        step(3, 1, 0, first_use=False)

        def body(g, carry):
            h = 2 * g
            step(h, 0, 1, first_use=False)
            step(h + 1, 1, 0, first_use=False)
            return carry

        lax.fori_loop(2, H // 2 - 1, body, 0)

        step(H - 2, 0, 1, first_use=False)
        wait_idx(1)
        fire_gathers(1)
        wait_gathers(0)
        wait_out(0)
        transpose(0)
        fire_out(H - 2, 0)
        wait_gathers(1)
        wait_out(1)
        transpose(1)
        fire_out(H - 1, 1)
        wait_out(0)
        wait_out(1)

    out = run(idx_t, table)
    return jnp.transpose(out, (2, 0, 1))


# speedup vs baseline: 1.8668x; 1.0001x over previous
import functools

import jax
import jax.numpy as jnp
from jax import lax
from jax.experimental import pallas as pl
from jax.experimental.pallas import tpu as pltpu
from jax.experimental.pallas import tpu_sc as plsc

_NC = 2
_NS = 16
_NW = _NC * _NS

_GRP = 128


def kernel(idxes, table):
    B, H = idxes.shape
    V, D = table.shape

    idx_t = idxes.T

    BW = B // _NW
    G = BW // _GRP
    LANES = 16
    assert BW * _NW == B and G * _GRP == BW and H % 2 == 0 and H >= 6

    mesh = plsc.VectorSubcoreMesh(core_axis_name="c", subcore_axis_name="s")

    @functools.partial(
        pl.kernel,
        mesh=mesh,
        out_type=jax.ShapeDtypeStruct((H * D, B), jnp.float32),
        compiler_params=pltpu.CompilerParams(
            use_tc_tiling_on_sc=False, needs_layout_passes=False),
        scratch_types=[
            pltpu.VMEM((G, _GRP), jnp.int32),
            pltpu.VMEM((G, _GRP), jnp.int32),
            pltpu.VMEM((BW, D), jnp.float32),
            pltpu.VMEM((BW, D), jnp.float32),
            pltpu.VMEM((D, BW), jnp.float32),
            pltpu.VMEM((D, BW), jnp.float32),
            pltpu.SemaphoreType.DMA,
            pltpu.SemaphoreType.DMA,
            pltpu.SemaphoreType.DMA,
            pltpu.SemaphoreType.DMA,
            pltpu.SemaphoreType.DMA,
            pltpu.SemaphoreType.DMA,
        ],
    )
    def run(idx_hbm, table_hbm, out_hbm,
            idx0, idx1, rows0, rows1, rt0, rt1,
            si0, si1, sg0, sg1, so0, so1):
        idx_v = (idx0, idx1)
        rows_v = (rows0, rows1)
        rt_v = (rt0, rt1)
        semi = (si0, si1)
        semg = (sg0, sg1)
        semo = (so0, so1)

        wid = lax.axis_index("s") * _NC + lax.axis_index("c")
        b0 = wid * BW

        def fire_idx(h, nb):
            for g in range(G):
                pltpu.async_copy(
                    idx_hbm.at[pl.ds(h, 1), pl.ds(b0 + g * _GRP, _GRP)],
                    idx_v[nb].at[pl.ds(g, 1)],
                    semi[nb])

        def wait_idx(nb):
            pltpu.make_async_copy(
                idx_hbm.at[pl.ds(0, G), pl.ds(0, _GRP)], idx_v[nb],
                semi[nb]).wait()

        def fire_gathers(nb):
            for g in range(G):
                pltpu.async_copy(
                    table_hbm.at[idx_v[nb].at[g]],
                    rows_v[nb].at[pl.ds(g * _GRP, _GRP)],
                    semg[nb])

        def wait_gathers(nb):
            pltpu.make_async_copy(
                table_hbm.at[pl.ds(0, BW)], rows_v[nb], semg[nb]).wait()

        lane_iota = jnp.arange(LANES, dtype=jnp.int32)
        col_ids = [jnp.full((LANES,), c, dtype=jnp.int32) for c in range(D)]

        def transpose(nb):
            rows = rows_v[nb]
            rt = rt_v[nb]

            @plsc.parallel_loop(0, BW // LANES)
            def tbody(lb):
                row_ids = lb * LANES + lane_iota
                for c in range(D):
                    v = plsc.load_gather(rows, [row_ids, col_ids[c]])
                    rt[c, pl.ds(lb * LANES, LANES)] = v

        def fire_out(h, nb):
            pltpu.async_copy(
                rt_v[nb], out_hbm.at[pl.ds(h * D, D), pl.ds(b0, BW)],
                semo[nb])

        def wait_out(nb):
            pltpu.make_async_copy(
                rt_v[nb], out_hbm.at[pl.ds(0, D), pl.ds(0, BW)],
                semo[nb]).wait()

        def step(h, b, pb, first_use):
            wait_idx(b)
            fire_gathers(b)
            wait_gathers(pb)
            if not first_use:
                wait_out(pb)
            transpose(pb)
            fire_out(h - 1, pb)
            fire_idx(h + 1, pb)

        fire_idx(0, 0)
        fire_idx(1, 1)
        wait_idx(0)
        fire_gathers(0)
        step(1, 1, 0, first_use=True)
        step(2, 0, 1, first_use=True)
        step(3, 1, 0, first_use=False)

        def body(g, carry):
            h = 2 * g
            step(h, 0, 1, first_use=False)
            step(h + 1, 1, 0, first_use=False)
            return carry

        lax.fori_loop(2, H // 2 - 1, body, 0)

        step(H - 2, 0, 1, first_use=False)
        wait_idx(1)
        fire_gathers(1)
        wait_gathers(0)
        wait_out(0)
        transpose(0)
        fire_out(H - 2, 0)
        wait_gathers(1)
        wait_out(1)
        transpose(1)
        fire_out(H - 1, 1)
        wait_out(0)
        wait_out(1)

    out = run(idx_t, table)
    return jnp.transpose(out.reshape(H, D, B), (2, 0, 1))
